# initial kernel scaffold (unmeasured)
import jax
import jax.numpy as jnp
from jax import lax
from jax.experimental import pallas as pl
from jax.experimental.pallas import tpu as pltpu


def kernel(
    x,
):
    def body(*refs):
        pass

    out_shape = jax.ShapeDtypeStruct(..., jnp.float32)
    return pl.pallas_call(body, out_shape=out_shape)(...)


# baseline (device time: 391347 ns/iter reference)
import jax
import jax.numpy as jnp
from jax import lax
from jax.experimental import pallas as pl
from jax.experimental.pallas import tpu as pltpu

M = 8192
N = 1024
HALF = M // 2


def kernel(x):
    def body(
        x_ref,
        out_ref,
        xloc,
        yrecv,
        load_sem,
        store_sem,
        ysend_sem,
        yrecv_sem,
        xsend_sem,
        xrecv_sem,
    ):
        my_x = lax.axis_index("x")
        my_y = lax.axis_index("y")

        barrier = pltpu.get_barrier_semaphore()
        for nbr in [(my_x, 1 - my_y), (1 - my_x, my_y)]:
            pl.semaphore_signal(
                barrier, inc=1, device_id=nbr,
                device_id_type=pl.DeviceIdType.MESH,
            )
        pl.semaphore_wait(barrier, 2)

        row0 = my_x * HALF
        rows = pl.ds(row0, HALF)

        yrdma = pltpu.make_async_remote_copy(
            src_ref=x_ref.at[rows, :],
            dst_ref=yrecv,
            send_sem=ysend_sem,
            recv_sem=yrecv_sem,
            device_id=(my_x, 1 - my_y),
            device_id_type=pl.DeviceIdType.MESH,
        )
        yrdma.start()

        load = pltpu.make_async_copy(x_ref.at[rows, :], xloc, load_sem)
        load.start()
        load.wait()
        yrdma.wait()

        xloc[:, :] = xloc[:, :] + yrecv[:, :]

        store = pltpu.make_async_copy(xloc, out_ref.at[rows, :], store_sem)
        store.start()
        xrdma = pltpu.make_async_remote_copy(
            src_ref=xloc,
            dst_ref=out_ref.at[rows, :],
            send_sem=xsend_sem,
            recv_sem=xrecv_sem,
            device_id=(1 - my_x, my_y),
            device_id_type=pl.DeviceIdType.MESH,
        )
        xrdma.start()
        store.wait()
        xrdma.wait()

    return pl.pallas_call(
        body,
        out_shape=jax.ShapeDtypeStruct((M, N), jnp.float32),
        in_specs=[pl.BlockSpec(memory_space=pl.ANY)],
        out_specs=pl.BlockSpec(memory_space=pl.ANY),
        scratch_shapes=[
            pltpu.VMEM((HALF, N), jnp.float32),
            pltpu.VMEM((HALF, N), jnp.float32),
            pltpu.SemaphoreType.DMA,
            pltpu.SemaphoreType.DMA,
            pltpu.SemaphoreType.DMA,
            pltpu.SemaphoreType.DMA,
            pltpu.SemaphoreType.DMA,
            pltpu.SemaphoreType.DMA,
        ],
        compiler_params=pltpu.CompilerParams(
            collective_id=0,
            vmem_limit_bytes=60 * 1024 * 1024,
        ),
    )(x)


# device time: 221951 ns/iter; 1.7632x vs baseline; 1.7632x over previous
import jax
import jax.numpy as jnp
from jax import lax
from jax.experimental import pallas as pl
from jax.experimental.pallas import tpu as pltpu

M = 8192
N = 1024
HALF = M // 2
C = 16
CH = HALF // C


def kernel(x):
    def body(
        x_ref,
        out_ref,
        xloc,
        yrecv,
        load_sem,
        store_sems,
        ysend_sems,
        yrecv_sems,
        xsend_sems,
        xrecv_sems,
    ):
        my_x = lax.axis_index("x")
        my_y = lax.axis_index("y")

        barrier = pltpu.get_barrier_semaphore()
        for nbr in [(my_x, 1 - my_y), (1 - my_x, my_y)]:
            pl.semaphore_signal(
                barrier, inc=1, device_id=nbr,
                device_id_type=pl.DeviceIdType.MESH,
            )
        pl.semaphore_wait(barrier, 2)

        row0 = my_x * HALF

        def yrdma(c):
            return pltpu.make_async_remote_copy(
                src_ref=x_ref.at[pl.ds(row0 + c * CH, CH), :],
                dst_ref=yrecv.at[pl.ds(c * CH, CH), :],
                send_sem=ysend_sems.at[c],
                recv_sem=yrecv_sems.at[c],
                device_id=(my_x, 1 - my_y),
                device_id_type=pl.DeviceIdType.MESH,
            )

        def xrdma(c):
            return pltpu.make_async_remote_copy(
                src_ref=xloc.at[pl.ds(c * CH, CH), :],
                dst_ref=out_ref.at[pl.ds(row0 + c * CH, CH), :],
                send_sem=xsend_sems.at[c],
                recv_sem=xrecv_sems.at[c],
                device_id=(1 - my_x, my_y),
                device_id_type=pl.DeviceIdType.MESH,
            )

        for c in range(C):
            yrdma(c).start()

        load = pltpu.make_async_copy(
            x_ref.at[pl.ds(row0, HALF), :], xloc, load_sem
        )
        load.start()
        load.wait()

        for c in range(C):
            chunk = pl.ds(c * CH, CH)
            yrdma(c).wait_recv()
            xloc[chunk, :] = xloc[chunk, :] + yrecv[chunk, :]
            pltpu.make_async_copy(
                xloc.at[chunk, :],
                out_ref.at[pl.ds(row0 + c * CH, CH), :],
                store_sems.at[c],
            ).start()
            xrdma(c).start()

        for c in range(C):
            pltpu.make_async_copy(
                xloc.at[pl.ds(c * CH, CH), :],
                out_ref.at[pl.ds(row0 + c * CH, CH), :],
                store_sems.at[c],
            ).wait()
            yrdma(c).wait_send()
            xr = xrdma(c)
            xr.wait_send()
            xr.wait_recv()

    return pl.pallas_call(
        body,
        out_shape=jax.ShapeDtypeStruct((M, N), jnp.float32),
        in_specs=[pl.BlockSpec(memory_space=pl.ANY)],
        out_specs=pl.BlockSpec(memory_space=pl.ANY),
        scratch_shapes=[
            pltpu.VMEM((HALF, N), jnp.float32),
            pltpu.VMEM((HALF, N), jnp.float32),
            pltpu.SemaphoreType.DMA,
            pltpu.SemaphoreType.DMA((C,)),
            pltpu.SemaphoreType.DMA((C,)),
            pltpu.SemaphoreType.DMA((C,)),
            pltpu.SemaphoreType.DMA((C,)),
            pltpu.SemaphoreType.DMA((C,)),
        ],
        compiler_params=pltpu.CompilerParams(
            collective_id=0,
            vmem_limit_bytes=60 * 1024 * 1024,
        ),
    )(x)


# device time: 216925 ns/iter; 1.8041x vs baseline; 1.0232x over previous
import jax
import jax.numpy as jnp
from jax import lax
from jax.experimental import pallas as pl
from jax.experimental.pallas import tpu as pltpu

M = 8192
N = 1024
HALF = M // 2
C = 32
CH = HALF // C


def kernel(x):
    def body(
        x_ref,
        out_ref,
        xloc,
        yrecv,
        load_sem,
        store_sems,
        ysend_sems,
        yrecv_sems,
        xsend_sems,
        xrecv_sems,
    ):
        my_x = lax.axis_index("x")
        my_y = lax.axis_index("y")

        barrier = pltpu.get_barrier_semaphore()
        for nbr in [(my_x, 1 - my_y), (1 - my_x, my_y)]:
            pl.semaphore_signal(
                barrier, inc=1, device_id=nbr,
                device_id_type=pl.DeviceIdType.MESH,
            )
        pl.semaphore_wait(barrier, 2)

        row0 = my_x * HALF

        def yrdma(c):
            return pltpu.make_async_remote_copy(
                src_ref=x_ref.at[pl.ds(row0 + c * CH, CH), :],
                dst_ref=yrecv.at[pl.ds(c * CH, CH), :],
                send_sem=ysend_sems.at[c],
                recv_sem=yrecv_sems.at[c],
                device_id=(my_x, 1 - my_y),
                device_id_type=pl.DeviceIdType.MESH,
            )

        def xrdma(c):
            return pltpu.make_async_remote_copy(
                src_ref=xloc.at[pl.ds(c * CH, CH), :],
                dst_ref=out_ref.at[pl.ds(row0 + c * CH, CH), :],
                send_sem=xsend_sems.at[c],
                recv_sem=xrecv_sems.at[c],
                device_id=(1 - my_x, my_y),
                device_id_type=pl.DeviceIdType.MESH,
            )

        for c in range(C):
            yrdma(c).start()

        load = pltpu.make_async_copy(
            x_ref.at[pl.ds(row0, HALF), :], xloc, load_sem
        )
        load.start()
        load.wait()

        for c in range(C):
            chunk = pl.ds(c * CH, CH)
            yrdma(c).wait_recv()
            xloc[chunk, :] = xloc[chunk, :] + yrecv[chunk, :]
            pltpu.make_async_copy(
                xloc.at[chunk, :],
                out_ref.at[pl.ds(row0 + c * CH, CH), :],
                store_sems.at[c],
            ).start()
            xrdma(c).start()

        for c in range(C):
            pltpu.make_async_copy(
                xloc.at[pl.ds(c * CH, CH), :],
                out_ref.at[pl.ds(row0 + c * CH, CH), :],
                store_sems.at[c],
            ).wait()
            yrdma(c).wait_send()
            xr = xrdma(c)
            xr.wait_send()
            xr.wait_recv()

    return pl.pallas_call(
        body,
        out_shape=jax.ShapeDtypeStruct((M, N), jnp.float32),
        in_specs=[pl.BlockSpec(memory_space=pl.ANY)],
        out_specs=pl.BlockSpec(memory_space=pl.ANY),
        scratch_shapes=[
            pltpu.VMEM((HALF, N), jnp.float32),
            pltpu.VMEM((HALF, N), jnp.float32),
            pltpu.SemaphoreType.DMA,
            pltpu.SemaphoreType.DMA((C,)),
            pltpu.SemaphoreType.DMA((C,)),
            pltpu.SemaphoreType.DMA((C,)),
            pltpu.SemaphoreType.DMA((C,)),
            pltpu.SemaphoreType.DMA((C,)),
        ],
        compiler_params=pltpu.CompilerParams(
            collective_id=0,
            vmem_limit_bytes=60 * 1024 * 1024,
        ),
    )(x)
